# baseline (speedup 1.0000x reference)
import functools

import jax
import jax.numpy as jnp
from jax import lax
from jax.experimental import pallas as pl
from jax.experimental.pallas import tpu as pltpu
from jax.experimental.pallas import tpu_sc as plsc

_N_USER = 10000
_N_ITEM = 40000
_N_NODES = _N_USER + _N_ITEM
_D = 64
_H = 32
_E = 800000
_B = 4096
_NB = 2 * _B

_N_TILES = 16
_CHUNK = 128
_E_PAD = 819200
_EDGES_PER_TILE = _E_PAD // _N_TILES
_N_CHUNKS = _EDGES_PER_TILE // _CHUNK
_ACC_ROWS = 50176
_ACC_PER_TILE = _ACC_ROWS // _N_TILES
_B_PER_TILE = _NB // _N_TILES
_B_CHUNKS = _B_PER_TILE // _CHUNK


def _sc_side_kernel():
    mesh = plsc.VectorSubcoreMesh(core_axis_name="c", subcore_axis_name="s")
    out_t = jax.ShapeDtypeStruct((_NB, _H), jnp.float32)

    @functools.partial(
        pl.kernel,
        mesh=mesh,
        out_type=[out_t, out_t, out_t, out_t],
        compiler_params=pltpu.CompilerParams(use_tc_tiling_on_sc=False),
        scratch_types=[
            pltpu.VMEM_SHARED((_ACC_ROWS, _H), jnp.float32),
            pltpu.VMEM((_CHUNK,), jnp.int32),
            pltpu.VMEM((_CHUNK,), jnp.int32),
            pltpu.VMEM((_CHUNK,), jnp.int32),
            pltpu.VMEM((_CHUNK,), jnp.int32),
            pltpu.VMEM((_CHUNK,), jnp.float32),
            pltpu.VMEM((_CHUNK,), jnp.float32),
            pltpu.VMEM((_CHUNK, _H), jnp.float32),
            pltpu.VMEM((_CHUNK, _H), jnp.float32),
            pltpu.VMEM((_CHUNK,), jnp.int32),
            pltpu.VMEM((_CHUNK, _H), jnp.float32),
            pltpu.SemaphoreType.DMA,
            pltpu.SemaphoreType.DMA,
            pltpu.SemaphoreType.DMA,
            pltpu.SemaphoreType.DMA,
            pltpu.SemaphoreType.DMA,
            pltpu.SemaphoreType.DMA,
            pltpu.SemaphoreType.DMA,
            pltpu.SemaphoreType.DMA,
            pltpu.SemaphoreType.DMA,
        ],
    )
    def sc(emb_lo, emb_hi, src_h, dst_h, val_h, zeros_h, bidx_h,
           side_lo, side_hi, embg_lo, embg_hi,
           acc, srcA, srcB, dstA, dstB, valA, valB, rowsA, rowsB,
           bidxv, gbuf, svA, svB, sdA, sdB, sgA, sgB, ssA, ssB, sem):
        cid = lax.axis_index("c")
        sid = lax.axis_index("s")

        pltpu.sync_copy(zeros_h.at[pl.ds(sid * _ACC_PER_TILE, _ACC_PER_TILE)],
                        acc.at[pl.ds(sid * _ACC_PER_TILE, _ACC_PER_TILE)])
        plsc.subcore_barrier()

        def edge_pass(emb_h):
            ebase = sid * _EDGES_PER_TILE

            def _off(i):
                return ebase + jnp.minimum(i, _N_CHUNKS - 1) * _CHUNK

            def sv_start(i, srcX, valX, svX):
                off = _off(i)
                pltpu.async_copy(src_h.at[pl.ds(off, _CHUNK)], srcX, svX)
                pltpu.async_copy(val_h.at[pl.ds(off, _CHUNK)], valX, svX)

            def sv_wait(srcX, valX, svX):
                pltpu.make_async_copy(src_h.at[pl.ds(0, _CHUNK)], srcX,
                                      svX).wait()
                pltpu.make_async_copy(val_h.at[pl.ds(0, _CHUNK)], valX,
                                      svX).wait()

            def dst_start(i, dstX, sdX):
                pltpu.async_copy(dst_h.at[pl.ds(_off(i), _CHUNK)], dstX, sdX)

            def dst_wait(dstX, sdX):
                pltpu.make_async_copy(dst_h.at[pl.ds(0, _CHUNK)], dstX,
                                      sdX).wait()

            def scatter_wait(rowsX, dstX, ssX):
                pltpu.make_async_copy(rowsX, acc.at[dstX], ssX).wait()

            _HC = _CHUNK // 2

            def gat_start(srcX, rowsX, sgX):
                pltpu.async_copy(emb_h.at[srcX.at[pl.ds(0, _HC)]],
                                 rowsX.at[pl.ds(0, _HC)], sgX)
                pltpu.async_copy(emb_h.at[srcX.at[pl.ds(_HC, _HC)]],
                                 rowsX.at[pl.ds(_HC, _HC)], sgX)

            def gat_wait(srcX, rowsX, sgX):
                pltpu.make_async_copy(emb_h.at[srcX.at[pl.ds(0, _HC)]],
                                      rowsX.at[pl.ds(0, _HC)], sgX).wait()
                pltpu.make_async_copy(emb_h.at[srcX.at[pl.ds(_HC, _HC)]],
                                      rowsX.at[pl.ds(_HC, _HC)], sgX).wait()

            def scale(valX, rowsX):
                @plsc.parallel_loop(0, _CHUNK, unroll=16)
                def _(e):
                    base16 = (e // 16) * 16
                    vv = valX[pl.ds(base16, 16)]
                    v16 = vv.at[jnp.full((16,), e - base16,
                                         dtype=jnp.int32)].get(
                                             mode="promise_in_bounds")
                    rowsX[e, pl.ds(0, 16)] = rowsX[e, pl.ds(0, 16)] * v16
                    rowsX[e, pl.ds(16, 16)] = rowsX[e, pl.ds(16, 16)] * v16

            def phase(i,
                      srcT, dstT, valT, rowsT, svT, sdT, sgT, ssT,
                      srcO, dstO, valO, rowsO, svO, sdO, sgO, ssO):
                sv_wait(srcO, valO, svO)
                scatter_wait(rowsO, dstO, ssO)
                dst_start(i + 1, dstO, sdO)
                gat_start(srcO, rowsO, sgO)
                gat_wait(srcT, rowsT, sgT)
                scale(valT, rowsT)
                dst_wait(dstT, sdT)
                pltpu.async_copy(rowsT, acc.at[dstT], ssT, add=True)
                sv_start(i + 2, srcT, valT, svT)

            sv_start(0, srcA, valA, svA)
            dst_start(0, dstA, sdA)
            sv_wait(srcA, valA, svA)
            gat_start(srcA, rowsA, sgA)
            sv_start(1, srcB, valB, svB)
            pltpu.async_copy(zeros_h.at[pl.ds(0, _CHUNK)], rowsB, ssB)

            def chunk_body(k, carry):
                i = 2 * k
                phase(i,
                      srcA, dstA, valA, rowsA, svA, sdA, sgA, ssA,
                      srcB, dstB, valB, rowsB, svB, sdB, sgB, ssB)
                phase(i + 1,
                      srcB, dstB, valB, rowsB, svB, sdB, sgB, ssB,
                      srcA, dstA, valA, rowsA, svA, sdA, sgA, ssA)
                return carry

            lax.fori_loop(0, _N_CHUNKS // 2, chunk_body, 0)
            scatter_wait(rowsB, dstB, ssB)
            gat_wait(srcA, rowsA, sgA)
            sv_wait(srcB, valB, svB)
            dst_wait(dstA, sdA)

        def epilogue(emb_h, side_o, embg_o):
            base = sid * _B_PER_TILE

            def g_body(j, carry):
                off = base + j * _CHUNK
                pltpu.sync_copy(bidx_h.at[pl.ds(off, _CHUNK)], bidxv)
                pltpu.sync_copy(acc.at[bidxv], gbuf)
                pltpu.sync_copy(gbuf, side_o.at[pl.ds(off, _CHUNK)])
                pltpu.async_copy(emb_h.at[bidxv], gbuf, sem).wait()
                pltpu.sync_copy(gbuf, embg_o.at[pl.ds(off, _CHUNK)])
                return carry

            lax.fori_loop(0, _B_CHUNKS, g_body, 0)

        @pl.when(cid == 0)
        def _():
            edge_pass(emb_lo)

        @pl.when(cid == 1)
        def _():
            edge_pass(emb_hi)

        plsc.subcore_barrier()

        @pl.when(cid == 0)
        def _():
            epilogue(emb_lo, side_lo, embg_lo)

        @pl.when(cid == 1)
        def _():
            epilogue(emb_hi, side_hi, embg_hi)

    return sc


def _tc_epilogue(eu_ref, ei_ref, su_ref, si_ref,
                 wg0, bg0, wb0, bb0, wg1, bg1, wb1, bb1, out_ref):
    eu = eu_ref[...]
    ei = ei_ref[...]
    su = su_ref[...]
    si = si_ref[...]
    g = jnp.sum(eu * ei, axis=1)
    for (wg, bg, wb, bb) in ((wg0, bg0, wb0, bb0), (wg1, bg1, wb1, bb1)):
        Wg = wg[...]
        Wb = wb[...]
        zu = (jnp.dot(su, Wg, preferred_element_type=jnp.float32) + bg[...]
              + jnp.dot(eu * su, Wb, preferred_element_type=jnp.float32)
              + bb[...])
        zi = (jnp.dot(si, Wg, preferred_element_type=jnp.float32) + bg[...]
              + jnp.dot(ei * si, Wb, preferred_element_type=jnp.float32)
              + bb[...])
        au = jnp.where(zu >= 0, zu, 0.2 * zu)
        ai = jnp.where(zi >= 0, zi, 0.2 * zi)
        nu = au / jnp.maximum(
            jnp.sqrt(jnp.sum(au * au, axis=1, keepdims=True)), 1e-12)
        ni = ai / jnp.maximum(
            jnp.sqrt(jnp.sum(ai * ai, axis=1, keepdims=True)), 1e-12)
        g = g + jnp.sum(nu * ni, axis=1)
    out_ref[...] = g


def kernel(adj_indices, adj_values, users, items, user_emb, item_emb,
           W_gc_0, b_gc_0, W_bi_0, b_bi_0, W_gc_1, b_gc_1, W_bi_1, b_bi_1):
    rate = 0.2
    drop_key = jax.random.key(12345)
    rt = (1.0 - rate) + jax.random.uniform(
        drop_key, (adj_values.shape[0],), dtype=jnp.float32)
    vals = adj_values * jnp.floor(rt) * (1.0 / (1.0 - rate))

    dst = adj_indices[0]
    src = adj_indices[1]
    pad = _E_PAD - _E
    srcp = jnp.concatenate([src, jnp.zeros((pad,), jnp.int32)])
    dstp = jnp.concatenate([dst, jnp.zeros((pad,), jnp.int32)])
    valp = jnp.concatenate([vals, jnp.zeros((pad,), jnp.float32)])

    emb = jnp.concatenate([user_emb, item_emb], axis=0)
    emb_lo = emb[:, :_H]
    emb_hi = emb[:, _H:]
    zeros_h = jnp.zeros((_ACC_ROWS, _H), jnp.float32)
    bidx = jnp.concatenate([users, items + _N_USER])

    side_lo, side_hi, embg_lo, embg_hi = _sc_side_kernel()(
        emb_lo, emb_hi, srcp, dstp, valp, zeros_h, bidx)

    side_g = jnp.concatenate([side_lo, side_hi], axis=1)
    emb_g = jnp.concatenate([embg_lo, embg_hi], axis=1)
    su, si = side_g[:_B], side_g[_B:]
    eu, ei = emb_g[:_B], emb_g[_B:]

    gamma = pl.pallas_call(
        _tc_epilogue,
        out_shape=jax.ShapeDtypeStruct((_B,), jnp.float32),
    )(eu, ei, su, si, W_gc_0, b_gc_0, W_bi_0, b_bi_0,
      W_gc_1, b_gc_1, W_bi_1, b_bi_1)
    return gamma


# speedup vs baseline: 1.1528x; 1.1528x over previous
import functools

import jax
import jax.numpy as jnp
import numpy as np
from jax import lax
from jax.experimental import pallas as pl
from jax.experimental.pallas import tpu as pltpu
from jax.experimental.pallas import tpu_sc as plsc

_N_USER = 10000
_N_ITEM = 40000
_N_NODES = _N_USER + _N_ITEM
_D = 64
_H = 32
_HP = 16
_E = 800000
_B = 4096
_NB = 2 * _B

_N_TILES = 16
_CHUNK = 128
_E_PAD = 819200
_EDGES_PER_TILE = _E_PAD // _N_TILES
_N_CHUNKS = _EDGES_PER_TILE // _CHUNK
_ACC_ROWS = 50176
_ACC_PER_TILE = _ACC_ROWS // _N_TILES
_B_PER_TILE = _NB // _N_TILES
_B_CHUNKS = _B_PER_TILE // _CHUNK

_INV_PERM = np.array(
    [32 * (k // 32) + 16 * (k % 2) + (k % 32) // 2 for k in range(64)],
    dtype=np.int32)


def _sc_side_kernel():
    mesh = plsc.VectorSubcoreMesh(core_axis_name="c", subcore_axis_name="s")

    @functools.partial(
        pl.kernel,
        mesh=mesh,
        out_type=[
            jax.ShapeDtypeStruct((_NB, _H), jnp.float32),
            jax.ShapeDtypeStruct((_NB, _H), jnp.float32),
            jax.ShapeDtypeStruct((_NB, _D), jnp.float32),
        ],
        compiler_params=pltpu.CompilerParams(
            use_tc_tiling_on_sc=False, needs_layout_passes=False),
        scratch_types=[
            pltpu.VMEM_SHARED((_ACC_ROWS, _H), jnp.float32),
            [pltpu.VMEM((_CHUNK,), jnp.int32)] * 2,
            [pltpu.VMEM((_CHUNK,), jnp.int32)] * 2,
            [pltpu.VMEM((_CHUNK,), jnp.float32)] * 2,
            [pltpu.VMEM((_CHUNK, _H), jnp.bfloat16)] * 2,
            [pltpu.VMEM((_CHUNK, _H), jnp.float32)] * 2,
            pltpu.VMEM((_CHUNK,), jnp.int32),
            pltpu.VMEM((_CHUNK, _H), jnp.float32),
            pltpu.VMEM((_CHUNK, _D), jnp.float32),
            [pltpu.SemaphoreType.DMA] * 2,
            [pltpu.SemaphoreType.DMA] * 2,
            [pltpu.SemaphoreType.DMA] * 2,
            [pltpu.SemaphoreType.DMA] * 2,
            pltpu.SemaphoreType.DMA,
        ],
    )
    def sc(emb16_lo, emb16_hi, src_h, dst_h, val_h, zeros_h, bidx_h,
           users_h, items_h, uemb_h, iemb_h,
           side_lo, side_hi, embg,
           acc, src2, dst2, val2, xi2, frows2,
           bidxv, gbuf, gbuf64, sv2, sd2, sg2, ss2, sem):
        cid = lax.axis_index("c")
        sid = lax.axis_index("s")

        pltpu.sync_copy(zeros_h.at[pl.ds(sid * _ACC_PER_TILE, _ACC_PER_TILE)],
                        acc.at[pl.ds(sid * _ACC_PER_TILE, _ACC_PER_TILE)])
        plsc.subcore_barrier()

        def edge_pass(emb_h):
            ebase = sid * _EDGES_PER_TILE

            def _off(i):
                return ebase + jnp.minimum(i, _N_CHUNKS - 1) * _CHUNK

            def sv_start(i, b):
                off = _off(i)
                pltpu.async_copy(src_h.at[pl.ds(off, _CHUNK)], src2[b],
                                 sv2[b])
                pltpu.async_copy(val_h.at[pl.ds(off, _CHUNK)], val2[b],
                                 sv2[b])

            def sv_wait(b):
                pltpu.make_async_copy(src_h.at[pl.ds(0, _CHUNK)], src2[b],
                                      sv2[b]).wait()
                pltpu.make_async_copy(val_h.at[pl.ds(0, _CHUNK)], val2[b],
                                      sv2[b]).wait()

            def dst_start(i, b):
                pltpu.async_copy(dst_h.at[pl.ds(_off(i), _CHUNK)], dst2[b],
                                 sd2[b])

            def dst_wait(b):
                pltpu.make_async_copy(dst_h.at[pl.ds(0, _CHUNK)], dst2[b],
                                      sd2[b]).wait()

            def gat_start(b):
                pltpu.async_copy(emb_h.at[src2[b]], xi2[b], sg2[b])

            def gat_wait(b):
                pltpu.make_async_copy(emb_h.at[src2[b]], xi2[b],
                                      sg2[b]).wait()

            def scatter_wait(b):
                pltpu.make_async_copy(frows2[b], acc.at[dst2[b]],
                                      ss2[b]).wait()

            def scale(b):
                valX = val2[b]
                xiX = xi2[b]
                frowsX = frows2[b]

                @plsc.parallel_loop(0, _CHUNK, unroll=16)
                def _(e):
                    base16 = (e // 16) * 16
                    vv = valX[pl.ds(base16, 16)]
                    v16 = vv.at[jnp.full((16,), e - base16,
                                         dtype=jnp.int32)].get(
                                             mode="promise_in_bounds")
                    xb = xiX[e, pl.ds(0, _H)]
                    ev, od = plsc.unpack(
                        xb, format=plsc.PackFormat.INTERLEAVED)
                    frowsX[e, pl.ds(0, 16)] = ev * v16
                    frowsX[e, pl.ds(16, 16)] = od * v16

            def phase(i, t, o):
                sv_wait(o)
                scatter_wait(o)
                dst_start(i + 1, o)
                gat_start(o)
                gat_wait(t)
                scale(t)
                dst_wait(t)
                pltpu.async_copy(frows2[t], acc.at[dst2[t]], ss2[t],
                                 add=True)
                sv_start(i + 2, t)

            sv_start(0, 0)
            dst_start(0, 0)
            sv_wait(0)
            gat_start(0)
            sv_start(1, 1)
            pltpu.async_copy(zeros_h.at[pl.ds(0, _CHUNK)], frows2[1], ss2[1])

            def chunk_body(k, carry):
                i = 2 * k
                phase(i, 0, 1)
                phase(i + 1, 1, 0)
                return carry

            lax.fori_loop(0, _N_CHUNKS // 2, chunk_body, 0)
            scatter_wait(1)
            gat_wait(0)
            sv_wait(1)
            dst_wait(0)

        def side_epilogue(side_o):
            base = sid * _B_PER_TILE

            def g_body(j, carry):
                off = base + j * _CHUNK
                pltpu.sync_copy(bidx_h.at[pl.ds(off, _CHUNK)], bidxv)
                pltpu.sync_copy(acc.at[bidxv], gbuf)
                pltpu.sync_copy(gbuf, side_o.at[pl.ds(off, _CHUNK)])
                return carry

            lax.fori_loop(0, _B_CHUNKS, g_body, 0)

        def ego_epilogue(tab_h, idx_h, out_base):
            def g_body(j, carry):
                off = sid * (_B // 8) + j * _CHUNK
                pltpu.sync_copy(idx_h.at[pl.ds(off, _CHUNK)], bidxv)
                pltpu.async_copy(tab_h.at[bidxv], gbuf64, sem).wait()
                pltpu.sync_copy(gbuf64,
                                embg.at[pl.ds(out_base + off, _CHUNK)])
                return carry

            lax.fori_loop(0, (_B // 8) // _CHUNK, g_body, 0)

        @pl.when(cid == 0)
        def _():
            edge_pass(emb16_lo)

        @pl.when(cid == 1)
        def _():
            edge_pass(emb16_hi)

        plsc.subcore_barrier()

        @pl.when(cid == 0)
        def _():
            side_epilogue(side_lo)

        @pl.when(cid == 1)
        def _():
            side_epilogue(side_hi)

        @pl.when(jnp.logical_and(cid == 0, sid < 8))
        def _():
            ego_epilogue(uemb_h, users_h, 0)

        @pl.when(jnp.logical_and(cid == 1, sid < 8))
        def _():
            ego_epilogue(iemb_h, items_h, _B)

    return sc


def _tc_epilogue(eu_ref, ei_ref, su_ref, si_ref,
                 wg0, bg0, wb0, bb0, wg1, bg1, wb1, bb1, out_ref):
    eu = eu_ref[...]
    ei = ei_ref[...]
    su = su_ref[...]
    si = si_ref[...]
    g = jnp.sum(eu * ei, axis=1)
    for (wg, bg, wb, bb) in ((wg0, bg0, wb0, bb0), (wg1, bg1, wb1, bb1)):
        Wg = wg[...]
        Wb = wb[...]
        zu = (jnp.dot(su, Wg, preferred_element_type=jnp.float32) + bg[...]
              + jnp.dot(eu * su, Wb, preferred_element_type=jnp.float32)
              + bb[...])
        zi = (jnp.dot(si, Wg, preferred_element_type=jnp.float32) + bg[...]
              + jnp.dot(ei * si, Wb, preferred_element_type=jnp.float32)
              + bb[...])
        au = jnp.where(zu >= 0, zu, 0.2 * zu)
        ai = jnp.where(zi >= 0, zi, 0.2 * zi)
        nu = au / jnp.maximum(
            jnp.sqrt(jnp.sum(au * au, axis=1, keepdims=True)), 1e-12)
        ni = ai / jnp.maximum(
            jnp.sqrt(jnp.sum(ai * ai, axis=1, keepdims=True)), 1e-12)
        g = g + jnp.sum(nu * ni, axis=1)
    out_ref[...] = g


def _pack_half(u_half, i_half):
    return jnp.concatenate([u_half, i_half], axis=0).astype(jnp.bfloat16)


def kernel(adj_indices, adj_values, users, items, user_emb, item_emb,
           W_gc_0, b_gc_0, W_bi_0, b_bi_0, W_gc_1, b_gc_1, W_bi_1, b_bi_1):
    rate = 0.2
    drop_key = jax.random.key(12345)
    rt = (1.0 - rate) + jax.random.uniform(
        drop_key, (adj_values.shape[0],), dtype=jnp.float32)
    vals = adj_values * jnp.floor(rt) * (1.0 / (1.0 - rate))

    dst = adj_indices[0]
    src = adj_indices[1]
    pad = _E_PAD - _E
    srcp = jnp.concatenate([src, jnp.zeros((pad,), jnp.int32)])
    dstp = jnp.concatenate([dst, jnp.zeros((pad,), jnp.int32)])
    valp = jnp.concatenate([vals, jnp.zeros((pad,), jnp.float32)])

    emb16_lo = _pack_half(user_emb[:, :_H], item_emb[:, :_H])
    emb16_hi = _pack_half(user_emb[:, _H:], item_emb[:, _H:])
    zeros_h = jnp.zeros((_ACC_ROWS, _H), jnp.float32)
    bidx = jnp.concatenate([users, items + _N_USER])

    side_lo, side_hi, emb_g = _sc_side_kernel()(
        emb16_lo, emb16_hi, srcp, dstp, valp, zeros_h, bidx,
        users, items, user_emb, item_emb)

    side_g = jnp.concatenate([side_lo, side_hi], axis=1)[:, _INV_PERM]
    su, si = side_g[:_B], side_g[_B:]
    eu, ei = emb_g[:_B], emb_g[_B:]

    gamma = pl.pallas_call(
        _tc_epilogue,
        out_shape=jax.ShapeDtypeStruct((_B,), jnp.float32),
    )(eu, ei, su, si, W_gc_0, b_gc_0, W_bi_0, b_bi_0,
      W_gc_1, b_gc_1, W_bi_1, b_bi_1)
    return gamma
